# baseline (speedup 1.0000x reference)
import functools

import jax
import jax.numpy as jnp
from jax import lax
from jax.experimental import pallas as pl
from jax.experimental.pallas import tpu as pltpu
from jax.experimental.pallas import tpu_sc as plsc

B = 1024
S1 = 25
S2 = 10
D = 128
DEG = 32
M = B * S1
N2 = M * S2

NC = 2
NS = 16
NW = NC * NS
BW = B // NW
MW = M // NW
NW2 = N2 // NW
CH = 80
L = 16


def _sc_gather(ids, features, adj128, c1, c2, t25, t10, nb):
    mesh = plsc.VectorSubcoreMesh(core_axis_name="c", subcore_axis_name="s")
    bw = nb // NW
    mw = bw * S1
    nw2 = mw * S2

    @functools.partial(
        pl.kernel,
        mesh=mesh,
        compiler_params=pltpu.CompilerParams(needs_layout_passes=False),
        out_type=[
            jax.ShapeDtypeStruct((nb, D), jnp.float32),
            jax.ShapeDtypeStruct((nb * S1, D), jnp.float32),
            jax.ShapeDtypeStruct((nb * S1, S2 * D), jnp.float32),
        ],
        scratch_types=[
            pltpu.VMEM((bw,), jnp.int32),
            pltpu.VMEM((bw,), jnp.int32),
            pltpu.VMEM((bw,), jnp.int32),
            pltpu.VMEM((bw, 4 * DEG), jnp.int32),
            pltpu.VMEM((mw,), jnp.int32),
            pltpu.VMEM((mw,), jnp.int32),
            pltpu.VMEM((mw,), jnp.int32),
            pltpu.VMEM((nw2,), jnp.int32),
            pltpu.VMEM((nw2,), jnp.int32),
            pltpu.VMEM((CH, 4 * DEG), jnp.int32),
            pltpu.VMEM((CH, 4 * DEG), jnp.int32),
            pltpu.VMEM((CH,), jnp.int32),
            pltpu.VMEM((CH,), jnp.int32),
            pltpu.VMEM((CH,), jnp.int32),
            pltpu.VMEM((CH,), jnp.int32),
            pltpu.VMEM((CH * S2,), jnp.int32),
            pltpu.VMEM((CH,), jnp.int32),
            pltpu.VMEM((CH,), jnp.int32),
            pltpu.VMEM((CH,), jnp.int32),
            pltpu.VMEM((CH,), jnp.int32),
            pltpu.VMEM((bw, D), jnp.float32),
            pltpu.VMEM((CH, D), jnp.float32),
            pltpu.VMEM((CH, D), jnp.float32),
            pltpu.VMEM((CH, D), jnp.float32),
            pltpu.VMEM((CH, D), jnp.float32),
            pltpu.SemaphoreType.DMA,
            pltpu.SemaphoreType.DMA,
            pltpu.SemaphoreType.DMA,
            pltpu.SemaphoreType.DMA,
            pltpu.SemaphoreType.DMA,
            pltpu.SemaphoreType.DMA,
            pltpu.SemaphoreType.DMA,
            pltpu.SemaphoreType.DMA,
            pltpu.SemaphoreType.DMA,
            pltpu.SemaphoreType.DMA,
            pltpu.SemaphoreType.DMA,
        ],
    )
    def body(ids_h, feat_h, adj_h, c1_h, c2_h, t25_h, t10_h,
             f0_h, f1_h, f2_h,
             ids_v, ro_v, off_v, adjr_v, c1_v, t25_v, ids1_v, c2_v, t10_v,
             adjca_v, adjcb_v, offma_v, offmb_v, aidxa_v, aidxb_v,
             ids2c_v, idx0_v, idx1_v, idx2_v, idx3_v,
             f0_v, row0_v, row1_v, row2_v, row3_v,
             sem, gs0, gs1, gs2, gs3, ws0, ws1, ws2, ws3, asa, asb):
        w = lax.axis_index("s") * NC + lax.axis_index("c")

        pltpu.sync_copy(ids_h.at[pl.ds(w * bw, bw)], ids_v)
        pltpu.sync_copy(c1_h.at[pl.ds(w * mw, mw)], c1_v)
        pltpu.sync_copy(t25_h, t25_v)
        pltpu.sync_copy(c2_h.at[pl.ds(w * nw2, nw2)], c2_v)
        pltpu.sync_copy(t10_h, t10_v)

        for i in range(max(bw // L, 1)):
            v = ids_v[pl.ds(i * L, L)]
            ro_v[pl.ds(i * L, L)] = v >> 2
            off_v[pl.ds(i * L, L)] = (v & 3) * DEG
        pltpu.async_copy(adj_h.at[ro_v], adjr_v, sem).wait()
        pltpu.async_copy(feat_h.at[ids_v], f0_v, sem).wait()
        pltpu.sync_copy(f0_v, f0_h.at[pl.ds(w * bw, bw)])

        for i in range(mw // L):
            bvec = t25_v[pl.ds(i * L, L)]
            cvec = plsc.load_gather(off_v, [bvec]) + c1_v[pl.ds(i * L, L)]
            ids1_v[pl.ds(i * L, L)] = plsc.load_gather(adjr_v, [bvec, cvec])

        NB = 4
        idxs = [idx0_v, idx1_v, idx2_v, idx3_v]
        rows = [row0_v, row1_v, row2_v, row3_v]
        gsems, wsems = [gs0, gs1, gs2, gs3], [ws0, ws1, ws2, ws3]

        def pipelined_gather(nt, fill_fn, dst_fn, src_fn):
            gh, wh = [None] * NB, [None] * NB

            def drain_gather(t):
                gh[t % NB].wait()
                wh[t % NB] = pltpu.async_copy(src_fn(rows[t % NB]),
                                              dst_fn(t), wsems[t % NB])

            for t in range(nt):
                cur = t % NB
                if wh[cur] is not None:
                    wh[cur].wait()
                    wh[cur] = None
                fill_fn(t, idxs[cur])
                gh[cur] = pltpu.async_copy(feat_h.at[idxs[cur]], rows[cur],
                                           gsems[cur])
                if t >= 2:
                    drain_gather(t - 2)
            for t in range(max(nt - 2, 0), nt):
                drain_gather(t)
            for t in range(max(nt - NB, 0), nt):
                if wh[t % NB] is not None:
                    wh[t % NB].wait()

        def f1_fill(t, idx_ref):
            for j in range(CH // L):
                idx_ref[pl.ds(j * L, L)] = ids1_v[pl.ds(t * CH + j * L, L)]

        pipelined_gather(
            mw // CH, f1_fill,
            lambda t: f1_h.at[pl.ds(w * mw + t * CH, CH)],
            lambda r: r)

        NK = mw // CH

        def adj_fill(k, aidx, aoff):
            for j in range(CH // L):
                mv = ids1_v[pl.ds(k * CH + j * L, L)]
                aidx[pl.ds(j * L, L)] = mv >> 2
                aoff[pl.ds(j * L, L)] = (mv & 3) * DEG

        def adj_drain(adjc, asem):
            pltpu.make_async_copy(adj_h.at[pl.ds(0, CH)], adjc, asem).wait()

        def f2_half(k, adjc, aidx, aoff, asem):
            adj_drain(adjc, asem)
            for j in range((CH * S2) // L):
                bvec = t10_v[pl.ds(k * CH * S2 + j * L, L)] - k * CH
                cvec = (plsc.load_gather(aoff, [bvec])
                        + c2_v[pl.ds(k * CH * S2 + j * L, L)])
                ids2c_v[pl.ds(j * L, L)] = plsc.load_gather(adjc,
                                                            [bvec, cvec])

            @pl.when(k + 2 < NK)
            def _():
                adj_fill(k + 2, aidx, aoff)
                pltpu.async_copy(adj_h.at[aidx], adjc, asem)

            def f2_fill(t, idx_ref):
                for j in range(CH // L):
                    idx_ref[pl.ds(j * L, L)] = ids2c_v[pl.ds(t * CH + j * L,
                                                             L)]

            pipelined_gather(
                S2, f2_fill,
                lambda t: f2_h.at[pl.ds(w * mw + k * CH + t * (CH // S2),
                                        CH // S2)],
                lambda r: r.reshape(CH // S2, S2 * D))

        adj_fill(0, aidxa_v, offma_v)
        pltpu.async_copy(adj_h.at[aidxa_v], adjca_v, asa)
        adj_fill(1, aidxb_v, offmb_v)
        pltpu.async_copy(adj_h.at[aidxb_v], adjcb_v, asb)

        def f2_pair(kk, _):
            f2_half(2 * kk, adjca_v, aidxa_v, offma_v, asa)
            f2_half(2 * kk + 1, adjcb_v, aidxb_v, offmb_v, asb)
            return 0

        lax.fori_loop(0, NK // 2, f2_pair, 0)
        if NK % 2:
            f2_half(NK - 1, adjca_v, aidxa_v, offma_v, asa)

    return body(ids, features, adj128, c1, c2, t25, t10)


def _tc_fused(f0, f1, f2r, w2s, b2s, w2nr, b2n, w1s, b1s, w1nr, b1n,
              fc_w, fc_b, nb):
    GB = 64
    ncls = fc_w.shape[1]

    def body(f0_r, f1_r, f2r_r, w2s_r, b2s_r, w2nr_r, b2n_r, w1s_r, b1s_r,
             w1nr_r, b1n_r, fcw_r, fcb_r, out_r):
        hs = jnp.dot(f1_r[...], w2s_r[...],
                     preferred_element_type=jnp.float32) + b2s_r[...]
        hn = jnp.dot(f2r_r[...], w2nr_r[...],
                     preferred_element_type=jnp.float32) + b2n_r[...]
        h2 = jnp.maximum(jnp.concatenate([hs, hn], axis=1), 0.0)
        h2r = h2.reshape(GB, S1 * 256)
        h1s = jnp.dot(f0_r[...], w1s_r[...],
                      preferred_element_type=jnp.float32) + b1s_r[...]
        h1n = jnp.dot(h2r, w1nr_r[...],
                      preferred_element_type=jnp.float32) + b1n_r[...]
        h1 = jnp.concatenate([h1s, h1n], axis=1)
        out_r[...] = jnp.dot(h1, fcw_r[...],
                             preferred_element_type=jnp.float32) + fcb_r[...]

    return pl.pallas_call(
        body,
        grid=(nb // GB,),
        in_specs=[
            pl.BlockSpec((GB, D), lambda i: (i, 0)),
            pl.BlockSpec((GB * S1, D), lambda i: (i, 0)),
            pl.BlockSpec((GB * S1, S2 * D), lambda i: (i, 0)),
            pl.BlockSpec((D, 128), lambda i: (0, 0)),
            pl.BlockSpec((1, 128), lambda i: (0, 0)),
            pl.BlockSpec((S2 * D, 128), lambda i: (0, 0)),
            pl.BlockSpec((1, 128), lambda i: (0, 0)),
            pl.BlockSpec((D, 128), lambda i: (0, 0)),
            pl.BlockSpec((1, 128), lambda i: (0, 0)),
            pl.BlockSpec((S1 * 256, 128), lambda i: (0, 0)),
            pl.BlockSpec((1, 128), lambda i: (0, 0)),
            pl.BlockSpec((256, ncls), lambda i: (0, 0)),
            pl.BlockSpec((1, ncls), lambda i: (0, 0)),
        ],
        out_specs=pl.BlockSpec((GB, ncls), lambda i: (i, 0)),
        out_shape=jax.ShapeDtypeStruct((nb, ncls), jnp.float32),
    )(f0, f1, f2r, w2s, b2s, w2nr, b2n, w1s, b1s, w1nr, b1n, fc_w, fc_b)


def _tc_layer2(f1, f2r, w2s, b2s, w2nr, b2n):
    blk = 1280

    def body(f1_r, f2r_r, w2s_r, b2s_r, w2nr_r, b2n_r, out_r):
        hs = jnp.dot(f1_r[...], w2s_r[...],
                     preferred_element_type=jnp.float32) + b2s_r[...]
        hn = jnp.dot(f2r_r[...], w2nr_r[...],
                     preferred_element_type=jnp.float32) + b2n_r[...]
        out_r[...] = jnp.maximum(jnp.concatenate([hs, hn], axis=1), 0.0)

    return pl.pallas_call(
        body,
        grid=(M // blk,),
        in_specs=[
            pl.BlockSpec((blk, D), lambda i: (i, 0)),
            pl.BlockSpec((blk, S2 * D), lambda i: (i, 0)),
            pl.BlockSpec((D, 128), lambda i: (0, 0)),
            pl.BlockSpec((1, 128), lambda i: (0, 0)),
            pl.BlockSpec((S2 * D, 128), lambda i: (0, 0)),
            pl.BlockSpec((1, 128), lambda i: (0, 0)),
        ],
        out_specs=pl.BlockSpec((blk, 256), lambda i: (i, 0)),
        out_shape=jax.ShapeDtypeStruct((M, 256), jnp.float32),
    )(f1, f2r, w2s, b2s, w2nr, b2n)


def _tc_layer1(f0, h2r, w1s, b1s, w1nr, b1n, fc_w, fc_b):
    blk = 256
    ncls = fc_w.shape[1]

    def body(f0_r, h2r_r, w1s_r, b1s_r, w1nr_r, b1n_r, fcw_r, fcb_r, out_r):
        hs = jnp.dot(f0_r[...], w1s_r[...],
                     preferred_element_type=jnp.float32) + b1s_r[...]
        hn = jnp.dot(h2r_r[...], w1nr_r[...],
                     preferred_element_type=jnp.float32) + b1n_r[...]
        h1 = jnp.concatenate([hs, hn], axis=1)
        out_r[...] = jnp.dot(h1, fcw_r[...],
                             preferred_element_type=jnp.float32) + fcb_r[...]

    return pl.pallas_call(
        body,
        grid=(B // blk,),
        in_specs=[
            pl.BlockSpec((blk, D), lambda i: (i, 0)),
            pl.BlockSpec((blk, S1 * 256), lambda i: (i, 0)),
            pl.BlockSpec((D, 128), lambda i: (0, 0)),
            pl.BlockSpec((1, 128), lambda i: (0, 0)),
            pl.BlockSpec((S1 * 256, 128), lambda i: (0, 0)),
            pl.BlockSpec((1, 128), lambda i: (0, 0)),
            pl.BlockSpec((256, ncls), lambda i: (0, 0)),
            pl.BlockSpec((1, ncls), lambda i: (0, 0)),
        ],
        out_specs=pl.BlockSpec((blk, ncls), lambda i: (i, 0)),
        out_shape=jax.ShapeDtypeStruct((B, ncls), jnp.float32),
    )(f0, h2r, w1s, b1s, w1nr, b1n, fc_w, fc_b)


def kernel(ids, features, adj, hop2_self_w, hop2_self_b, hop1_self_w,
           hop1_self_b, hop2_neib_w, hop2_neib_b, hop1_neib_w, hop1_neib_b,
           fc_w, fc_b):
    c1 = jax.random.randint(jax.random.key(1), (B * S1,), 0, DEG,
                            dtype=jnp.int32)
    c2 = jax.random.randint(jax.random.key(2), (M * S2,), 0, DEG,
                            dtype=jnp.int32)
    t25 = jnp.arange((B // 2 // NW) * S1, dtype=jnp.int32) // S1
    t10 = jnp.arange((B // 2 // NW) * S1 * S2, dtype=jnp.int32) // S2

    adj128 = adj.astype(jnp.int32).reshape(-1, 4 * DEG)

    w2nr = jnp.repeat(hop2_neib_w, S2, axis=0) * (1.0 / S2)
    w1nr = jnp.repeat(hop1_neib_w, S1, axis=0) * (1.0 / S1)

    nh = B // 2
    ids32 = ids.astype(jnp.int32)
    outs = []
    for h in range(2):
        f0, f1, f2r = _sc_gather(
            ids32[h * nh:(h + 1) * nh], features, adj128,
            c1[h * nh * S1:(h + 1) * nh * S1],
            c2[h * nh * S1 * S2:(h + 1) * nh * S1 * S2],
            t25, t10, nh)
        outs.append(_tc_fused(f0, f1, f2r,
                              hop2_self_w, hop2_self_b.reshape(1, -1),
                              w2nr, hop2_neib_b.reshape(1, -1),
                              hop1_self_w, hop1_self_b.reshape(1, -1),
                              w1nr, hop1_neib_b.reshape(1, -1),
                              fc_w, fc_b.reshape(1, -1), nh))
    return jnp.concatenate(outs, axis=0)


# speedup vs baseline: 1.0329x; 1.0329x over previous
import functools

import jax
import jax.numpy as jnp
from jax import lax
from jax.experimental import pallas as pl
from jax.experimental.pallas import tpu as pltpu
from jax.experimental.pallas import tpu_sc as plsc

B = 1024
S1 = 25
S2 = 10
D = 128
DEG = 32
M = B * S1
N2 = M * S2

NC = 2
NS = 16
NW = NC * NS
BW = B // NW
MW = M // NW
NW2 = N2 // NW
CH = 80
L = 16


def _sc_gather(ids, features, adj128, c1, c2, t25, t10):
    mesh = plsc.VectorSubcoreMesh(core_axis_name="c", subcore_axis_name="s")

    @functools.partial(
        pl.kernel,
        mesh=mesh,
        compiler_params=pltpu.CompilerParams(needs_layout_passes=False),
        out_type=[
            jax.ShapeDtypeStruct((B, D), jnp.float32),
            jax.ShapeDtypeStruct((M, D), jnp.float32),
            jax.ShapeDtypeStruct((M, S2 * D), jnp.float32),
        ],
        scratch_types=[
            pltpu.VMEM((BW,), jnp.int32),
            pltpu.VMEM((BW,), jnp.int32),
            pltpu.VMEM((BW,), jnp.int32),
            pltpu.VMEM((BW, 4 * DEG), jnp.int32),
            pltpu.VMEM((MW,), jnp.int32),
            pltpu.VMEM((MW,), jnp.int32),
            pltpu.VMEM((MW,), jnp.int32),
            pltpu.VMEM((NW2,), jnp.int32),
            pltpu.VMEM((NW2,), jnp.int32),
            pltpu.VMEM((CH, 4 * DEG), jnp.int32),
            pltpu.VMEM((CH, 4 * DEG), jnp.int32),
            pltpu.VMEM((CH,), jnp.int32),
            pltpu.VMEM((CH,), jnp.int32),
            pltpu.VMEM((CH,), jnp.int32),
            pltpu.VMEM((CH,), jnp.int32),
            pltpu.VMEM((CH * S2,), jnp.int32),
            pltpu.VMEM((CH,), jnp.int32),
            pltpu.VMEM((CH,), jnp.int32),
            pltpu.VMEM((CH,), jnp.int32),
            pltpu.VMEM((CH,), jnp.int32),
            pltpu.VMEM((BW, D), jnp.float32),
            pltpu.VMEM((CH, D), jnp.float32),
            pltpu.VMEM((CH, D), jnp.float32),
            pltpu.VMEM((CH, D), jnp.float32),
            pltpu.VMEM((CH, D), jnp.float32),
            pltpu.SemaphoreType.DMA,
            pltpu.SemaphoreType.DMA,
            pltpu.SemaphoreType.DMA,
            pltpu.SemaphoreType.DMA,
            pltpu.SemaphoreType.DMA,
            pltpu.SemaphoreType.DMA,
            pltpu.SemaphoreType.DMA,
            pltpu.SemaphoreType.DMA,
            pltpu.SemaphoreType.DMA,
            pltpu.SemaphoreType.DMA,
            pltpu.SemaphoreType.DMA,
        ],
    )
    def body(ids_h, feat_h, adj_h, c1_h, c2_h, t25_h, t10_h,
             f0_h, f1_h, f2_h,
             ids_v, ro_v, off_v, adjr_v, c1_v, t25_v, ids1_v, c2_v, t10_v,
             adjca_v, adjcb_v, offma_v, offmb_v, aidxa_v, aidxb_v,
             ids2c_v, idx0_v, idx1_v, idx2_v, idx3_v,
             f0_v, row0_v, row1_v, row2_v, row3_v,
             sem, gs0, gs1, gs2, gs3, ws0, ws1, ws2, ws3, asa, asb):
        w = lax.axis_index("s") * NC + lax.axis_index("c")

        pltpu.sync_copy(ids_h.at[pl.ds(w * BW, BW)], ids_v)
        pltpu.sync_copy(c1_h.at[pl.ds(w * MW, MW)], c1_v)
        pltpu.sync_copy(t25_h, t25_v)
        pltpu.sync_copy(c2_h.at[pl.ds(w * NW2, NW2)], c2_v)
        pltpu.sync_copy(t10_h, t10_v)

        for i in range(BW // L):
            v = ids_v[pl.ds(i * L, L)]
            ro_v[pl.ds(i * L, L)] = v >> 2
            off_v[pl.ds(i * L, L)] = (v & 3) * DEG
        pltpu.async_copy(adj_h.at[ro_v], adjr_v, sem).wait()
        pltpu.async_copy(feat_h.at[ids_v], f0_v, sem).wait()
        pltpu.sync_copy(f0_v, f0_h.at[pl.ds(w * BW, BW)])

        for i in range(MW // L):
            bvec = t25_v[pl.ds(i * L, L)]
            cvec = plsc.load_gather(off_v, [bvec]) + c1_v[pl.ds(i * L, L)]
            ids1_v[pl.ds(i * L, L)] = plsc.load_gather(adjr_v, [bvec, cvec])

        NB = 4
        idxs = [idx0_v, idx1_v, idx2_v, idx3_v]
        rows = [row0_v, row1_v, row2_v, row3_v]
        gsems, wsems = [gs0, gs1, gs2, gs3], [ws0, ws1, ws2, ws3]

        def pipelined_gather(nt, fill_fn, dst_fn, src_fn):
            gh, wh = [None] * NB, [None] * NB

            def drain_gather(t):
                gh[t % NB].wait()
                wh[t % NB] = pltpu.async_copy(src_fn(rows[t % NB]),
                                              dst_fn(t), wsems[t % NB])

            for t in range(nt):
                cur = t % NB
                if wh[cur] is not None:
                    wh[cur].wait()
                    wh[cur] = None
                fill_fn(t, idxs[cur])
                gh[cur] = pltpu.async_copy(feat_h.at[idxs[cur]], rows[cur],
                                           gsems[cur])
                if t >= 2:
                    drain_gather(t - 2)
            for t in range(max(nt - 2, 0), nt):
                drain_gather(t)
            for t in range(max(nt - NB, 0), nt):
                if wh[t % NB] is not None:
                    wh[t % NB].wait()

        def f1_fill(t, idx_ref):
            for j in range(CH // L):
                idx_ref[pl.ds(j * L, L)] = ids1_v[pl.ds(t * CH + j * L, L)]

        pipelined_gather(
            MW // CH, f1_fill,
            lambda t: f1_h.at[pl.ds(w * MW + t * CH, CH)],
            lambda r: r)

        NK = MW // CH

        def adj_fill(k, aidx, aoff):
            for j in range(CH // L):
                mv = ids1_v[pl.ds(k * CH + j * L, L)]
                aidx[pl.ds(j * L, L)] = mv >> 2
                aoff[pl.ds(j * L, L)] = (mv & 3) * DEG

        def adj_drain(adjc, asem):
            pltpu.make_async_copy(adj_h.at[pl.ds(0, CH)], adjc, asem).wait()

        def f2_half(k, adjc, aidx, aoff, asem):
            adj_drain(adjc, asem)
            for j in range((CH * S2) // L):
                bvec = t10_v[pl.ds(k * CH * S2 + j * L, L)] - k * CH
                cvec = (plsc.load_gather(aoff, [bvec])
                        + c2_v[pl.ds(k * CH * S2 + j * L, L)])
                ids2c_v[pl.ds(j * L, L)] = plsc.load_gather(adjc,
                                                            [bvec, cvec])

            @pl.when(k + 2 < NK)
            def _():
                adj_fill(k + 2, aidx, aoff)
                pltpu.async_copy(adj_h.at[aidx], adjc, asem)

            def f2_fill(t, idx_ref):
                for j in range(CH // L):
                    idx_ref[pl.ds(j * L, L)] = ids2c_v[pl.ds(t * CH + j * L,
                                                             L)]

            pipelined_gather(
                S2, f2_fill,
                lambda t: f2_h.at[pl.ds(w * MW + k * CH + t * (CH // S2),
                                        CH // S2)],
                lambda r: r.reshape(CH // S2, S2 * D))

        adj_fill(0, aidxa_v, offma_v)
        pltpu.async_copy(adj_h.at[aidxa_v], adjca_v, asa)
        adj_fill(1, aidxb_v, offmb_v)
        pltpu.async_copy(adj_h.at[aidxb_v], adjcb_v, asb)

        def f2_pair(kk, _):
            f2_half(2 * kk, adjca_v, aidxa_v, offma_v, asa)
            f2_half(2 * kk + 1, adjcb_v, aidxb_v, offmb_v, asb)
            return 0

        lax.fori_loop(0, NK // 2, f2_pair, 0)

    return body(ids, features, adj128, c1, c2, t25, t10)


def _tc_fused(f0, f1, f2r, w2s, b2s, w2nr, b2n, w1s, b1s, w1nr, b1n,
              fc_w, fc_b):
    GB = 64
    ncls = fc_w.shape[1]

    def body(f0_r, f1_r, f2r_r, w2s_r, b2s_r, w2nr_r, b2n_r, w1s_r, b1s_r,
             w1nr_r, b1n_r, fcw_r, fcb_r, out_r):
        hs = jnp.dot(f1_r[...], w2s_r[...],
                     preferred_element_type=jnp.float32) + b2s_r[...]
        hn = jnp.dot(f2r_r[...], w2nr_r[...],
                     preferred_element_type=jnp.float32) + b2n_r[...]
        h2 = jnp.maximum(jnp.concatenate([hs, hn], axis=1), 0.0)
        h2r = h2.reshape(GB, S1 * 256)
        h1s = jnp.dot(f0_r[...], w1s_r[...],
                      preferred_element_type=jnp.float32) + b1s_r[...]
        h1n = jnp.dot(h2r, w1nr_r[...],
                      preferred_element_type=jnp.float32) + b1n_r[...]
        h1 = jnp.concatenate([h1s, h1n], axis=1)
        out_r[...] = jnp.dot(h1, fcw_r[...],
                             preferred_element_type=jnp.float32) + fcb_r[...]

    return pl.pallas_call(
        body,
        grid=(B // GB,),
        in_specs=[
            pl.BlockSpec((GB, D), lambda i: (i, 0)),
            pl.BlockSpec((GB * S1, D), lambda i: (i, 0)),
            pl.BlockSpec((GB * S1, S2 * D), lambda i: (i, 0)),
            pl.BlockSpec((D, 128), lambda i: (0, 0)),
            pl.BlockSpec((1, 128), lambda i: (0, 0)),
            pl.BlockSpec((S2 * D, 128), lambda i: (0, 0)),
            pl.BlockSpec((1, 128), lambda i: (0, 0)),
            pl.BlockSpec((D, 128), lambda i: (0, 0)),
            pl.BlockSpec((1, 128), lambda i: (0, 0)),
            pl.BlockSpec((S1 * 256, 128), lambda i: (0, 0)),
            pl.BlockSpec((1, 128), lambda i: (0, 0)),
            pl.BlockSpec((256, ncls), lambda i: (0, 0)),
            pl.BlockSpec((1, ncls), lambda i: (0, 0)),
        ],
        out_specs=pl.BlockSpec((GB, ncls), lambda i: (i, 0)),
        out_shape=jax.ShapeDtypeStruct((B, ncls), jnp.float32),
    )(f0, f1, f2r, w2s, b2s, w2nr, b2n, w1s, b1s, w1nr, b1n, fc_w, fc_b)


def kernel(ids, features, adj, hop2_self_w, hop2_self_b, hop1_self_w,
           hop1_self_b, hop2_neib_w, hop2_neib_b, hop1_neib_w, hop1_neib_b,
           fc_w, fc_b):
    c1 = jax.random.randint(jax.random.key(1), (B * S1,), 0, DEG,
                            dtype=jnp.int32)
    c2 = jax.random.randint(jax.random.key(2), (M * S2,), 0, DEG,
                            dtype=jnp.int32)
    t25 = jnp.arange(MW, dtype=jnp.int32) // S1
    t10 = jnp.arange(NW2, dtype=jnp.int32) // S2

    adj128 = adj.astype(jnp.int32).reshape(-1, 4 * DEG)
    f0, f1, f2r = _sc_gather(ids.astype(jnp.int32), features,
                             adj128, c1, c2, t25, t10)

    w2nr = jnp.repeat(hop2_neib_w, S2, axis=0) * (1.0 / S2)
    w1nr = jnp.repeat(hop1_neib_w, S1, axis=0) * (1.0 / S1)

    return _tc_fused(f0, f1, f2r,
                     hop2_self_w, hop2_self_b.reshape(1, -1),
                     w2nr, hop2_neib_b.reshape(1, -1),
                     hop1_self_w, hop1_self_b.reshape(1, -1),
                     w1nr, hop1_neib_b.reshape(1, -1),
                     fc_w, fc_b.reshape(1, -1))
